# baseline (speedup 1.0000x reference)
import functools

import jax
import jax.numpy as jnp
from jax import lax
from jax.experimental import pallas as pl
from jax.experimental.pallas import tpu as pltpu
from jax.experimental.pallas import tpu_sc as plsc

N = 10000
E = 320000
F = 128
NRBF = 20
CUTOFF = 5.0

NC = 2
NS = 16
NW = NC * NS
EPW = E // NW
ACC_N = 10112
RPT = ACC_N // NS
FB = 32

BC = 32
BV = 32


def _mlp_body(x_ref, w1_ref, b1_ref, w2_ref, b2_ref, o_ref):
    h = jnp.dot(x_ref[...], w1_ref[...], preferred_element_type=jnp.float32)
    h = h + b1_ref[...]
    h = h * jax.nn.sigmoid(h)
    o_ref[...] = jnp.dot(h, w2_ref[...], preferred_element_type=jnp.float32) \
        + b2_ref[...]


def _mlp(node_scalar, W1, b1, W2, b2):
    BN = 1000
    grid = (N // BN,)
    return pl.pallas_call(
        _mlp_body,
        grid=grid,
        in_specs=[
            pl.BlockSpec((BN, F), lambda i: (i, 0)),
            pl.BlockSpec((F, F), lambda i: (0, 0)),
            pl.BlockSpec((1, F), lambda i: (0, 0)),
            pl.BlockSpec((F, 3 * F), lambda i: (0, 0)),
            pl.BlockSpec((1, 3 * F), lambda i: (0, 0)),
        ],
        out_specs=pl.BlockSpec((BN, 3 * F), lambda i: (i, 0)),
        out_shape=jax.ShapeDtypeStruct((N, 3 * F), jnp.float32),
    )(node_scalar, W1, b1.reshape(1, F), W2, b2.reshape(1, 3 * F))


def _rbf_body(dist_ref, diff_ref, wr_ref, br_ref,
              o1_ref, o2_ref, o3_ref, s0_ref, s1_ref, s2_ref):
    r = dist_ref[...]
    n_vals = lax.broadcasted_iota(jnp.int32, (1, NRBF), 1).astype(jnp.float32) + 1.0
    facs = jnp.concatenate(
        [n_vals * (jnp.pi / CUTOFF),
         jnp.full((1, 1), jnp.pi / CUTOFF, jnp.float32)], axis=1)
    offs = jnp.concatenate(
        [jnp.zeros((1, NRBF), jnp.float32),
         jnp.full((1, 1), 0.5 * jnp.pi, jnp.float32)], axis=1)
    sall = jnp.sin(r * facs + offs)
    sinc = sall[:, :NRBF] / r
    cosq = sall[:, NRBF:]
    rbf = jnp.dot(sinc, wr_ref[...], preferred_element_type=jnp.float32)
    rbf = rbf + br_ref[...]
    cc = jnp.where(r < CUTOFF, 0.5 * (cosq + 1.0), 0.0)
    rbf = rbf * cc
    o1_ref[...] = rbf[:, :F]
    o2_ref[...] = rbf[:, F:2 * F]
    o3_ref[...] = rbf[:, 2 * F:]
    rstd = diff_ref[...] / r
    s0_ref[...] = rstd[:, 0:1]
    s1_ref[...] = rstd[:, 1:2]
    s2_ref[...] = rstd[:, 2:3]


def _rbf(edge_dist, edge_diff, Wr, br):
    BE = 4000
    grid = (E // BE,)
    out = jax.ShapeDtypeStruct((E, F), jnp.float32)
    out1 = jax.ShapeDtypeStruct((E, 1), jnp.float32)
    return pl.pallas_call(
        _rbf_body,
        grid=grid,
        in_specs=[
            pl.BlockSpec((BE, 1), lambda i: (i, 0)),
            pl.BlockSpec((BE, 3), lambda i: (i, 0)),
            pl.BlockSpec((NRBF, 3 * F), lambda i: (0, 0)),
            pl.BlockSpec((1, 3 * F), lambda i: (0, 0)),
        ],
        out_specs=[pl.BlockSpec((BE, F), lambda i: (i, 0))] * 3
        + [pl.BlockSpec((BE, 1), lambda i: (i, 0))] * 3,
        out_shape=[out, out, out, out1, out1, out1],
    )(edge_dist.reshape(E, 1), edge_diff, Wr, br.reshape(1, 3 * F))


_SC_MESH = plsc.VectorSubcoreMesh(core_axis_name="c", subcore_axis_name="s")


def _zero_acc(buf0, acc, s):
    buf = buf0.at[pl.ds(0, FB)]
    zrow = jnp.zeros((16,), jnp.float32)

    def zbody(i, _):
        for g in range(F // 16):
            buf0[i, pl.ds(g * 16, 16)] = zrow
        return 0

    lax.fori_loop(0, FB, zbody, 0)
    base = s * RPT
    for k in range(RPT // FB):
        pltpu.sync_copy(buf, acc.at[pl.ds(base + k * FB, FB)])
    rem = RPT % FB
    if rem:
        pltpu.sync_copy(buf0.at[pl.ds(0, rem)],
                        acc.at[pl.ds(base + (RPT // FB) * FB, rem)])


def _flush_acc(buf0, acc, out, c, s):
    buf = buf0.at[pl.ds(0, FB)]
    base = s * RPT
    for k in range(RPT // FB):
        pltpu.sync_copy(acc.at[pl.ds(base + k * FB, FB)], buf)
        pltpu.sync_copy(buf, out.at[c, pl.ds(base + k * FB, FB)])
    rem = RPT % FB
    if rem:
        pltpu.sync_copy(acc.at[pl.ds(base + (RPT // FB) * FB, rem)],
                        buf0.at[pl.ds(0, rem)])
        pltpu.sync_copy(buf0.at[pl.ds(0, rem)],
                        out.at[c, pl.ds(base + (RPT // FB) * FB, rem)])


def _edge_pipeline(cfg, src1d, sets, isb, sems, combine, wid):
    B = cfg["B"]
    base = wid * (cfg["mcpw"] * B)
    dst1d = cfg.get("dst")
    acc = cfg.get("acc")

    def issue_idx(p, e0):
        pltpu.async_copy(src1d.at[pl.ds(e0, B)], isb[p], sems["i"][p])

    def wait_idx(p, e0):
        pltpu.make_async_copy(src1d.at[pl.ds(e0, B)], isb[p], sems["i"][p]).wait()

    def issue_set(p, e0):
        st = sets[p]
        if dst1d is not None:
            pltpu.async_copy(dst1d.at[pl.ds(e0, B)], st["id"], sems["d"][p])
        for tab, buf in st["gath"]:
            pltpu.async_copy(tab.at[isb[p]], buf, sems["d"][p])
        for lin, buf in st["lin"]:
            pltpu.async_copy(lin.at[pl.ds(e0, B)], buf, sems["d"][p])

    def wait_set(p, e0):
        st = sets[p]
        if dst1d is not None:
            pltpu.make_async_copy(dst1d.at[pl.ds(e0, B)], st["id"],
                                  sems["d"][p]).wait()
        for tab, buf in st["gath"]:
            pltpu.make_async_copy(tab.at[isb[p]], buf, sems["d"][p]).wait()
        for lin, buf in st["lin"]:
            pltpu.make_async_copy(lin.at[pl.ds(e0, B)], buf,
                                  sems["d"][p]).wait()

    def issue_out(p, e0):
        st = sets[p]
        if acc is not None:
            pltpu.async_copy(st["msg"], acc.at[st["id"]], sems["s"][p],
                             add=True)
        for hbm, buf in st["sto"]:
            pltpu.sync_copy(buf, hbm.at[pl.ds(e0, B)])

    def wait_out(p, e0):
        st = sets[p]
        if acc is not None:
            pltpu.make_async_copy(st["msg"], acc.at[st["id"]],
                                  sems["s"][p]).wait()

    pltpu.sync_copy(src1d.at[pl.ds(base, B)], isb[0])
    issue_set(0, base)
    issue_idx(1, base + B)

    def pair(jj, _):
        e0 = base + jj * (2 * B)
        e1 = e0 + B
        e2 = e1 + B
        e3 = e2 + B
        wait_idx(1, e1)

        @pl.when(jj > 0)
        def _():
            wait_out(1, e1 - 2 * B)

        issue_set(1, e1)
        wait_set(0, e0)
        issue_idx(0, e2)
        combine(0, B)
        issue_out(0, e0)
        wait_idx(0, e2)
        wait_out(0, e0)
        issue_set(0, e2)
        wait_set(1, e1)
        issue_idx(1, e3)
        combine(1, B)
        issue_out(1, e1)
        return 0

    lax.fori_loop(0, cfg["pairs"], pair, 0)

    wait_set(0, base + cfg["mcpw"] * B)
    wait_idx(1, base + (cfg["mcpw"] + 1) * B)
    wait_out(1, base + (cfg["mcpw"] - 1) * B)

    T = cfg["tail"]
    if T:
        its, itd = cfg["tails"]
        e0t = cfg["main"] + wid * T
        st = sets[0]
        pltpu.sync_copy(src1d.at[pl.ds(e0t, T)], its)
        if dst1d is not None:
            pltpu.sync_copy(dst1d.at[pl.ds(e0t, T)], itd)
        for tab, buf in st["gath"]:
            pltpu.sync_copy(tab.at[its], buf.at[pl.ds(0, T)])
        for lin, buf in st["lin"]:
            pltpu.sync_copy(lin.at[pl.ds(e0t, T)], buf.at[pl.ds(0, T)])
        combine(0, T)
        if acc is not None:
            pltpu.sync_copy(st["msg"].at[pl.ds(0, T)], acc.at[itd], add=True)
        for hbm, buf in st["sto"]:
            pltpu.sync_copy(buf.at[pl.ds(0, T)], hbm.at[pl.ds(e0t, T)])


_CS_MCPW = EPW // BC
_CS_CFGC = dict(B=BC, mcpw=_CS_MCPW, pairs=_CS_MCPW // 2,
                tail=EPW - _CS_MCPW * BC, main=NW * _CS_MCPW * BC)
_V_MCPW = EPW // BV
_V_CFGC = dict(B=BV, mcpw=_V_MCPW, pairs=_V_MCPW // 2,
               tail=EPW - _V_MCPW * BV, main=NW * _V_MCPW * BV)

_EFT = jax.ShapeDtypeStruct((E, F), jnp.float32)
_PART = jax.ShapeDtypeStruct((NC, ACC_N, F), jnp.float32)


@functools.partial(
    pl.kernel,
    out_type=[_EFT, _EFT, _PART],
    mesh=_SC_MESH,
    scratch_types=[
        pltpu.VMEM((BC,), jnp.int32), pltpu.VMEM((BC,), jnp.int32),
        pltpu.VMEM((BC,), jnp.int32), pltpu.VMEM((BC,), jnp.int32),
        pltpu.VMEM((16,), jnp.int32), pltpu.VMEM((16,), jnp.int32),
        pltpu.VMEM((BC, 3 * F), jnp.float32), pltpu.VMEM((BC, 3 * F), jnp.float32),
        pltpu.VMEM((BC, F), jnp.float32), pltpu.VMEM((BC, F), jnp.float32),
        pltpu.VMEM((BC, F), jnp.float32), pltpu.VMEM((BC, F), jnp.float32),
        pltpu.VMEM((BC, F), jnp.float32), pltpu.VMEM((BC, F), jnp.float32),
        pltpu.VMEM_SHARED((ACC_N, F), jnp.float32),
        pltpu.SemaphoreType.DMA, pltpu.SemaphoreType.DMA,
        pltpu.SemaphoreType.DMA, pltpu.SemaphoreType.DMA,
        pltpu.SemaphoreType.DMA, pltpu.SemaphoreType.DMA,
    ],
)
def _cspass(src1d, dst1d, a123, r1, r2, r3, c1o, c3o, out,
            isa, isbb, ida, idb, its, itd,
            ga, gb, l1a, l1b, l2a, l2b, l3a, l3b, acc,
            sda, sdb, sia, sib, ssa, ssb):
    c = lax.axis_index("c")
    s = lax.axis_index("s")
    wid = c * NS + s

    _zero_acc(l1a, acc, s)
    plsc.subcore_barrier()

    sets = [
        {"id": ida, "gath": [(a123, ga)],
         "lin": [(r1, l1a), (r2, l2a), (r3, l3a)], "msg": l2a,
         "sto": [(c1o, l1a), (c3o, l3a)]},
        {"id": idb, "gath": [(a123, gb)],
         "lin": [(r1, l1b), (r2, l2b), (r3, l3b)], "msg": l2b,
         "sto": [(c1o, l1b), (c3o, l3b)]},
    ]
    sems = {"d": [sda, sdb], "i": [sia, sib], "s": [ssa, ssb]}
    gbufs = [(ga, l1a, l2a, l3a), (gb, l1b, l2b, l3b)]

    def combine(p, nrows):
        g, l1b_, l2b_, l3b_ = gbufs[p]

        def rbody(i, _):
            for gi in range(F // 16):
                sl = pl.ds(gi * 16, 16)
                s2 = pl.ds(F + gi * 16, 16)
                s3 = pl.ds(2 * F + gi * 16, 16)
                l1b_[i, sl] = g[i, sl] * l1b_[i, sl]
                l2b_[i, sl] = g[i, s2] * l2b_[i, sl]
                l3b_[i, sl] = g[i, s3] * l3b_[i, sl]
            return 0

        lax.fori_loop(0, nrows, rbody, 0)

    cfg = dict(_CS_CFGC, dst=dst1d, acc=acc, tails=(its, itd))
    _edge_pipeline(cfg, src1d, sets, [isa, isbb], sems, combine, wid)
    plsc.subcore_barrier()
    _flush_acc(l1a, acc, out, c, s)


@functools.partial(
    pl.kernel,
    out_type=_PART,
    mesh=_SC_MESH,
    scratch_types=[
        pltpu.VMEM((BV,), jnp.int32), pltpu.VMEM((BV,), jnp.int32),
        pltpu.VMEM((BV,), jnp.int32), pltpu.VMEM((BV,), jnp.int32),
        pltpu.VMEM((16,), jnp.int32), pltpu.VMEM((16,), jnp.int32),
        pltpu.VMEM((BV, F), jnp.float32), pltpu.VMEM((BV, F), jnp.float32),
        pltpu.VMEM((BV, F), jnp.float32), pltpu.VMEM((BV, F), jnp.float32),
        pltpu.VMEM((BV, F), jnp.float32), pltpu.VMEM((BV, F), jnp.float32),
        pltpu.VMEM((BV + 16,), jnp.float32), pltpu.VMEM((BV + 16,), jnp.float32),
        pltpu.VMEM_SHARED((ACC_N, F), jnp.float32),
        pltpu.SemaphoreType.DMA, pltpu.SemaphoreType.DMA,
        pltpu.SemaphoreType.DMA, pltpu.SemaphoreType.DMA,
        pltpu.SemaphoreType.DMA, pltpu.SemaphoreType.DMA,
    ],
)
def _vpass(src1d, dst1d, tabn, c1l, c3l, rstdl, out,
           isa, isbb, ida, idb, its, itd,
           gna, gnb, l1a, l1b, l3a, l3b, ra, rb, acc,
           sda, sdb, sia, sib, ssa, ssb):
    c = lax.axis_index("c")
    s = lax.axis_index("s")
    wid = c * NS + s

    _zero_acc(gna, acc, s)
    plsc.subcore_barrier()

    sets = [
        {"id": ida, "gath": [(tabn, gna)],
         "lin": [(c1l, l1a), (c3l, l3a), (rstdl, ra.at[pl.ds(0, BV)])],
         "msg": gna, "sto": []},
        {"id": idb, "gath": [(tabn, gnb)],
         "lin": [(c1l, l1b), (c3l, l3b), (rstdl, rb.at[pl.ds(0, BV)])],
         "msg": gnb, "sto": []},
    ]
    sems = {"d": [sda, sdb], "i": [sia, sib], "s": [ssa, ssb]}
    gbufs = [(gna, l1a, l3a, ra), (gnb, l1b, l3b, rb)]

    def combine(p, nrows):
        gn, l1b_, l3b_, rr = gbufs[p]

        def gbody(k, _):
            i0 = k * 16
            mv = rr[pl.ds(i0, 16)]
            for r in range(16):
                m = mv[r]
                for gi in range(F // 16):
                    sl = pl.ds(gi * 16, 16)
                    gn[i0 + r, sl] = gn[i0 + r, sl] * l1b_[i0 + r, sl] \
                        + l3b_[i0 + r, sl] * m
            return 0

        lax.fori_loop(0, nrows // 16, gbody, 0)

    cfg = dict(_V_CFGC, dst=dst1d, acc=acc, tails=(its, itd))
    _edge_pipeline(cfg, src1d, sets, [isa, isbb], sems, combine, wid)
    plsc.subcore_barrier()
    _flush_acc(gna, acc, out, c, s)


def _final_body(ns_ref, nv_ref, ps_ref, p0_ref, p1_ref, p2_ref,
                os_ref, ov_ref):
    os_ref[...] = ns_ref[...] + ps_ref[0] + ps_ref[1]
    dv0 = p0_ref[0] + p0_ref[1]
    dv1 = p1_ref[0] + p1_ref[1]
    dv2 = p2_ref[0] + p2_ref[1]
    ov_ref[...] = nv_ref[...] + jnp.stack([dv0, dv1, dv2], axis=1)


def _final(node_scalar, node_vector, ps, pv0, pv1, pv2):
    BN = 1000
    grid = (N // BN,)
    part_spec = pl.BlockSpec((NC, BN, F), lambda i: (0, i, 0))
    return pl.pallas_call(
        _final_body,
        grid=grid,
        in_specs=[
            pl.BlockSpec((BN, F), lambda i: (i, 0)),
            pl.BlockSpec((BN, 3, F), lambda i: (i, 0, 0)),
            part_spec, part_spec, part_spec, part_spec,
        ],
        out_specs=[
            pl.BlockSpec((BN, F), lambda i: (i, 0)),
            pl.BlockSpec((BN, 3, F), lambda i: (i, 0, 0)),
        ],
        out_shape=[
            jax.ShapeDtypeStruct((N, F), jnp.float32),
            jax.ShapeDtypeStruct((N, 3, F), jnp.float32),
        ],
    )(node_scalar, node_vector, ps, pv0, pv1, pv2)


def kernel(node_scalar, node_vector, edge_index, edge_diff, edge_dist,
           W1, b1, W2, b2, Wr, br):
    src1d = edge_index[0]
    dst1d = edge_index[1]

    a123 = _mlp(node_scalar, W1, b1, W2, b2)
    r1, r2, r3, s0, s1, s2 = _rbf(edge_dist, edge_diff, Wr, br)
    rstd = [s0.reshape(E), s1.reshape(E), s2.reshape(E)]

    nvs = [node_vector[:, 0, :], node_vector[:, 1, :], node_vector[:, 2, :]]

    c1, c3, ps = _cspass(src1d, dst1d, a123, r1, r2, r3)
    pv = [_vpass(src1d, dst1d, nvs[d], c1, c3, rstd[d]) for d in range(3)]

    out_s, out_v = _final(node_scalar, node_vector, ps, pv[0], pv[1], pv[2])
    return (out_s, out_v)


# speedup vs baseline: 1.0056x; 1.0056x over previous
import functools

import jax
import jax.numpy as jnp
from jax import lax
from jax.experimental import pallas as pl
from jax.experimental.pallas import tpu as pltpu
from jax.experimental.pallas import tpu_sc as plsc

N = 10000
E = 320000
F = 128
NRBF = 20
CUTOFF = 5.0

NC = 2
NS = 16
NW = NC * NS
EPW = E // NW
ACC_N = 10112
RPT = ACC_N // NS
FB = 32

BC = 32
BV = 32


def _mlp_body(x_ref, w1_ref, b1_ref, w2_ref, b2_ref, o_ref):
    h = jnp.dot(x_ref[...], w1_ref[...], preferred_element_type=jnp.float32)
    h = h + b1_ref[...]
    h = h * jax.nn.sigmoid(h)
    o_ref[...] = jnp.dot(h, w2_ref[...], preferred_element_type=jnp.float32) \
        + b2_ref[...]


def _mlp(node_scalar, W1, b1, W2, b2):
    BN = 1000
    grid = (N // BN,)
    return pl.pallas_call(
        _mlp_body,
        grid=grid,
        in_specs=[
            pl.BlockSpec((BN, F), lambda i: (i, 0)),
            pl.BlockSpec((F, F), lambda i: (0, 0)),
            pl.BlockSpec((1, F), lambda i: (0, 0)),
            pl.BlockSpec((F, 3 * F), lambda i: (0, 0)),
            pl.BlockSpec((1, 3 * F), lambda i: (0, 0)),
        ],
        out_specs=pl.BlockSpec((BN, 3 * F), lambda i: (i, 0)),
        out_shape=jax.ShapeDtypeStruct((N, 3 * F), jnp.float32),
    )(node_scalar, W1, b1.reshape(1, F), W2, b2.reshape(1, 3 * F))


def _rbf_body(dist_ref, diff_ref, wr_ref, br_ref,
              o1_ref, o2_ref, o3_ref, s0_ref, s1_ref, s2_ref):
    r = dist_ref[...]
    n_vals = lax.broadcasted_iota(jnp.int32, (1, NRBF), 1).astype(jnp.float32) + 1.0
    facs = jnp.concatenate(
        [n_vals * (jnp.pi / CUTOFF),
         jnp.full((1, 1), jnp.pi / CUTOFF, jnp.float32)], axis=1)
    offs = jnp.concatenate(
        [jnp.zeros((1, NRBF), jnp.float32),
         jnp.full((1, 1), 0.5 * jnp.pi, jnp.float32)], axis=1)
    sall = jnp.sin(r * facs + offs)
    sinc = sall[:, :NRBF] / r
    cosq = sall[:, NRBF:]
    rbf = jnp.dot(sinc, wr_ref[...], preferred_element_type=jnp.float32)
    rbf = rbf + br_ref[...]
    cc = jnp.where(r < CUTOFF, 0.5 * (cosq + 1.0), 0.0)
    rbf = rbf * cc
    o1_ref[...] = rbf[:, :F]
    o2_ref[...] = rbf[:, F:2 * F]
    o3_ref[...] = rbf[:, 2 * F:]
    rstd = diff_ref[...] / r
    s0_ref[...] = rstd[:, 0:1]
    s1_ref[...] = rstd[:, 1:2]
    s2_ref[...] = rstd[:, 2:3]


def _rbf(edge_dist, edge_diff, Wr, br):
    BE = 4000
    grid = (E // BE,)
    out = jax.ShapeDtypeStruct((E, F), jnp.float32)
    out1 = jax.ShapeDtypeStruct((E, 1), jnp.float32)
    return pl.pallas_call(
        _rbf_body,
        grid=grid,
        in_specs=[
            pl.BlockSpec((BE, 1), lambda i: (i, 0)),
            pl.BlockSpec((BE, 3), lambda i: (i, 0)),
            pl.BlockSpec((NRBF, 3 * F), lambda i: (0, 0)),
            pl.BlockSpec((1, 3 * F), lambda i: (0, 0)),
        ],
        out_specs=[pl.BlockSpec((BE, F), lambda i: (i, 0))] * 3
        + [pl.BlockSpec((BE, 1), lambda i: (i, 0))] * 3,
        out_shape=[out, out, out, out1, out1, out1],
    )(edge_dist.reshape(E, 1), edge_diff, Wr, br.reshape(1, 3 * F))


_SC_MESH = plsc.VectorSubcoreMesh(core_axis_name="c", subcore_axis_name="s")


def _zero_acc(buf0, acc, s):
    buf = buf0.at[pl.ds(0, FB)]
    zrow = jnp.zeros((16,), jnp.float32)

    def zbody(i, _):
        for g in range(F // 16):
            buf0[i, pl.ds(g * 16, 16)] = zrow
        return 0

    lax.fori_loop(0, FB, zbody, 0)
    base = s * RPT
    for k in range(RPT // FB):
        pltpu.sync_copy(buf, acc.at[pl.ds(base + k * FB, FB)])
    rem = RPT % FB
    if rem:
        pltpu.sync_copy(buf0.at[pl.ds(0, rem)],
                        acc.at[pl.ds(base + (RPT // FB) * FB, rem)])


def _flush_acc(buf0, acc, out, c, s):
    buf = buf0.at[pl.ds(0, FB)]
    base = s * RPT
    for k in range(RPT // FB):
        pltpu.sync_copy(acc.at[pl.ds(base + k * FB, FB)], buf)
        pltpu.sync_copy(buf, out.at[c, pl.ds(base + k * FB, FB)])
    rem = RPT % FB
    if rem:
        pltpu.sync_copy(acc.at[pl.ds(base + (RPT // FB) * FB, rem)],
                        buf0.at[pl.ds(0, rem)])
        pltpu.sync_copy(buf0.at[pl.ds(0, rem)],
                        out.at[c, pl.ds(base + (RPT // FB) * FB, rem)])


def _edge_pipeline(cfg, src1d, sets, isb, sems, combine, wid):
    B = cfg["B"]
    base = wid * (cfg["mcpw"] * B)
    dst1d = cfg.get("dst")
    acc = cfg.get("acc")

    def issue_idx(p, e0):
        pltpu.async_copy(src1d.at[pl.ds(e0, B)], isb[p], sems["i"][p])

    def wait_idx(p, e0):
        pltpu.make_async_copy(src1d.at[pl.ds(e0, B)], isb[p], sems["i"][p]).wait()

    def issue_set(p, e0):
        st = sets[p]
        if dst1d is not None:
            pltpu.async_copy(dst1d.at[pl.ds(e0, B)], st["id"], sems["d"][p])
        for tab, buf in st["gath"]:
            pltpu.async_copy(tab.at[isb[p]], buf, sems["d"][p])
        for lin, buf in st["lin"]:
            pltpu.async_copy(lin.at[pl.ds(e0, B)], buf, sems["d"][p])

    def wait_set(p, e0):
        st = sets[p]
        if dst1d is not None:
            pltpu.make_async_copy(dst1d.at[pl.ds(e0, B)], st["id"],
                                  sems["d"][p]).wait()
        for tab, buf in st["gath"]:
            pltpu.make_async_copy(tab.at[isb[p]], buf, sems["d"][p]).wait()
        for lin, buf in st["lin"]:
            pltpu.make_async_copy(lin.at[pl.ds(e0, B)], buf,
                                  sems["d"][p]).wait()

    def issue_out(p, e0):
        st = sets[p]
        if acc is not None:
            pltpu.async_copy(st["msg"], acc.at[st["id"]], sems["s"][p],
                             add=True)
        for hbm, buf in st["sto"]:
            pltpu.async_copy(buf, hbm.at[pl.ds(e0, B)], sems["t"][p])

    def wait_out(p, e0):
        st = sets[p]
        if acc is not None:
            pltpu.make_async_copy(st["msg"], acc.at[st["id"]],
                                  sems["s"][p]).wait()
        for hbm, buf in st["sto"]:
            pltpu.make_async_copy(hbm.at[pl.ds(e0, B)], buf,
                                  sems["t"][p]).wait()

    pltpu.sync_copy(src1d.at[pl.ds(base, B)], isb[0])
    issue_set(0, base)
    issue_idx(1, base + B)

    def pair(jj, _):
        e0 = base + jj * (2 * B)
        e1 = e0 + B
        e2 = e1 + B
        e3 = e2 + B
        wait_idx(1, e1)

        @pl.when(jj > 0)
        def _():
            wait_out(1, e1 - 2 * B)

        issue_set(1, e1)
        wait_set(0, e0)
        issue_idx(0, e2)
        combine(0, B)
        issue_out(0, e0)
        wait_idx(0, e2)
        wait_out(0, e0)
        issue_set(0, e2)
        wait_set(1, e1)
        issue_idx(1, e3)
        combine(1, B)
        issue_out(1, e1)
        return 0

    lax.fori_loop(0, cfg["pairs"], pair, 0)

    wait_set(0, base + cfg["mcpw"] * B)
    wait_idx(1, base + (cfg["mcpw"] + 1) * B)
    wait_out(1, base + (cfg["mcpw"] - 1) * B)

    T = cfg["tail"]
    if T:
        its, itd = cfg["tails"]
        e0t = cfg["main"] + wid * T
        st = sets[0]
        pltpu.sync_copy(src1d.at[pl.ds(e0t, T)], its)
        if dst1d is not None:
            pltpu.sync_copy(dst1d.at[pl.ds(e0t, T)], itd)
        for tab, buf in st["gath"]:
            pltpu.sync_copy(tab.at[its], buf.at[pl.ds(0, T)])
        for lin, buf in st["lin"]:
            pltpu.sync_copy(lin.at[pl.ds(e0t, T)], buf.at[pl.ds(0, T)])
        combine(0, T)
        if acc is not None:
            pltpu.sync_copy(st["msg"].at[pl.ds(0, T)], acc.at[itd], add=True)
        for hbm, buf in st["sto"]:
            pltpu.sync_copy(buf.at[pl.ds(0, T)], hbm.at[pl.ds(e0t, T)])


_CS_MCPW = EPW // BC
_CS_CFGC = dict(B=BC, mcpw=_CS_MCPW, pairs=_CS_MCPW // 2,
                tail=EPW - _CS_MCPW * BC, main=NW * _CS_MCPW * BC)
_V_MCPW = EPW // BV
_V_CFGC = dict(B=BV, mcpw=_V_MCPW, pairs=_V_MCPW // 2,
               tail=EPW - _V_MCPW * BV, main=NW * _V_MCPW * BV)

_EFT = jax.ShapeDtypeStruct((E, F), jnp.float32)
_PART = jax.ShapeDtypeStruct((NC, ACC_N, F), jnp.float32)


@functools.partial(
    pl.kernel,
    out_type=[_EFT, _EFT, _PART],
    mesh=_SC_MESH,
    scratch_types=[
        pltpu.VMEM((BC,), jnp.int32), pltpu.VMEM((BC,), jnp.int32),
        pltpu.VMEM((BC,), jnp.int32), pltpu.VMEM((BC,), jnp.int32),
        pltpu.VMEM((16,), jnp.int32), pltpu.VMEM((16,), jnp.int32),
        pltpu.VMEM((BC, 3 * F), jnp.float32), pltpu.VMEM((BC, 3 * F), jnp.float32),
        pltpu.VMEM((BC, F), jnp.float32), pltpu.VMEM((BC, F), jnp.float32),
        pltpu.VMEM((BC, F), jnp.float32), pltpu.VMEM((BC, F), jnp.float32),
        pltpu.VMEM((BC, F), jnp.float32), pltpu.VMEM((BC, F), jnp.float32),
        pltpu.VMEM_SHARED((ACC_N, F), jnp.float32),
        pltpu.SemaphoreType.DMA, pltpu.SemaphoreType.DMA,
        pltpu.SemaphoreType.DMA, pltpu.SemaphoreType.DMA,
        pltpu.SemaphoreType.DMA, pltpu.SemaphoreType.DMA,
        pltpu.SemaphoreType.DMA, pltpu.SemaphoreType.DMA,
    ],
)
def _cspass(src1d, dst1d, a123, r1, r2, r3, c1o, c3o, out,
            isa, isbb, ida, idb, its, itd,
            ga, gb, l1a, l1b, l2a, l2b, l3a, l3b, acc,
            sda, sdb, sia, sib, ssa, ssb, sta, stb):
    c = lax.axis_index("c")
    s = lax.axis_index("s")
    wid = c * NS + s

    _zero_acc(l1a, acc, s)
    plsc.subcore_barrier()

    sets = [
        {"id": ida, "gath": [(a123, ga)],
         "lin": [(r1, l1a), (r2, l2a), (r3, l3a)], "msg": l2a,
         "sto": [(c1o, l1a), (c3o, l3a)]},
        {"id": idb, "gath": [(a123, gb)],
         "lin": [(r1, l1b), (r2, l2b), (r3, l3b)], "msg": l2b,
         "sto": [(c1o, l1b), (c3o, l3b)]},
    ]
    sems = {"d": [sda, sdb], "i": [sia, sib], "s": [ssa, ssb],
            "t": [sta, stb]}
    gbufs = [(ga, l1a, l2a, l3a), (gb, l1b, l2b, l3b)]

    def combine(p, nrows):
        g, l1b_, l2b_, l3b_ = gbufs[p]

        def rbody(i, _):
            for gi in range(F // 16):
                sl = pl.ds(gi * 16, 16)
                s2 = pl.ds(F + gi * 16, 16)
                s3 = pl.ds(2 * F + gi * 16, 16)
                l1b_[i, sl] = g[i, sl] * l1b_[i, sl]
                l2b_[i, sl] = g[i, s2] * l2b_[i, sl]
                l3b_[i, sl] = g[i, s3] * l3b_[i, sl]
            return 0

        lax.fori_loop(0, nrows, rbody, 0)

    cfg = dict(_CS_CFGC, dst=dst1d, acc=acc, tails=(its, itd))
    _edge_pipeline(cfg, src1d, sets, [isa, isbb], sems, combine, wid)
    plsc.subcore_barrier()
    _flush_acc(l1a, acc, out, c, s)


@functools.partial(
    pl.kernel,
    out_type=_PART,
    mesh=_SC_MESH,
    scratch_types=[
        pltpu.VMEM((BV,), jnp.int32), pltpu.VMEM((BV,), jnp.int32),
        pltpu.VMEM((BV,), jnp.int32), pltpu.VMEM((BV,), jnp.int32),
        pltpu.VMEM((16,), jnp.int32), pltpu.VMEM((16,), jnp.int32),
        pltpu.VMEM((BV, F), jnp.float32), pltpu.VMEM((BV, F), jnp.float32),
        pltpu.VMEM((BV, F), jnp.float32), pltpu.VMEM((BV, F), jnp.float32),
        pltpu.VMEM((BV, F), jnp.float32), pltpu.VMEM((BV, F), jnp.float32),
        pltpu.VMEM((BV + 16,), jnp.float32), pltpu.VMEM((BV + 16,), jnp.float32),
        pltpu.VMEM_SHARED((ACC_N, F), jnp.float32),
        pltpu.SemaphoreType.DMA, pltpu.SemaphoreType.DMA,
        pltpu.SemaphoreType.DMA, pltpu.SemaphoreType.DMA,
        pltpu.SemaphoreType.DMA, pltpu.SemaphoreType.DMA,
    ],
)
def _vpass(src1d, dst1d, tabn, c1l, c3l, rstdl, out,
           isa, isbb, ida, idb, its, itd,
           gna, gnb, l1a, l1b, l3a, l3b, ra, rb, acc,
           sda, sdb, sia, sib, ssa, ssb):
    c = lax.axis_index("c")
    s = lax.axis_index("s")
    wid = c * NS + s

    _zero_acc(gna, acc, s)
    plsc.subcore_barrier()

    sets = [
        {"id": ida, "gath": [(tabn, gna)],
         "lin": [(c1l, l1a), (c3l, l3a), (rstdl, ra.at[pl.ds(0, BV)])],
         "msg": gna, "sto": []},
        {"id": idb, "gath": [(tabn, gnb)],
         "lin": [(c1l, l1b), (c3l, l3b), (rstdl, rb.at[pl.ds(0, BV)])],
         "msg": gnb, "sto": []},
    ]
    sems = {"d": [sda, sdb], "i": [sia, sib], "s": [ssa, ssb]}
    gbufs = [(gna, l1a, l3a, ra), (gnb, l1b, l3b, rb)]

    def combine(p, nrows):
        gn, l1b_, l3b_, rr = gbufs[p]

        def gbody(k, _):
            i0 = k * 16
            mv = rr[pl.ds(i0, 16)]
            for r in range(16):
                m = mv[r]
                for gi in range(F // 16):
                    sl = pl.ds(gi * 16, 16)
                    gn[i0 + r, sl] = gn[i0 + r, sl] * l1b_[i0 + r, sl] \
                        + l3b_[i0 + r, sl] * m
            return 0

        lax.fori_loop(0, nrows // 16, gbody, 0)

    cfg = dict(_V_CFGC, dst=dst1d, acc=acc, tails=(its, itd))
    _edge_pipeline(cfg, src1d, sets, [isa, isbb], sems, combine, wid)
    plsc.subcore_barrier()
    _flush_acc(gna, acc, out, c, s)


def _final_body(ns_ref, nv_ref, ps_ref, p0_ref, p1_ref, p2_ref,
                os_ref, ov_ref):
    os_ref[...] = ns_ref[...] + ps_ref[0] + ps_ref[1]
    dv0 = p0_ref[0] + p0_ref[1]
    dv1 = p1_ref[0] + p1_ref[1]
    dv2 = p2_ref[0] + p2_ref[1]
    ov_ref[...] = nv_ref[...] + jnp.stack([dv0, dv1, dv2], axis=1)


def _final(node_scalar, node_vector, ps, pv0, pv1, pv2):
    BN = 1000
    grid = (N // BN,)
    part_spec = pl.BlockSpec((NC, BN, F), lambda i: (0, i, 0))
    return pl.pallas_call(
        _final_body,
        grid=grid,
        in_specs=[
            pl.BlockSpec((BN, F), lambda i: (i, 0)),
            pl.BlockSpec((BN, 3, F), lambda i: (i, 0, 0)),
            part_spec, part_spec, part_spec, part_spec,
        ],
        out_specs=[
            pl.BlockSpec((BN, F), lambda i: (i, 0)),
            pl.BlockSpec((BN, 3, F), lambda i: (i, 0, 0)),
        ],
        out_shape=[
            jax.ShapeDtypeStruct((N, F), jnp.float32),
            jax.ShapeDtypeStruct((N, 3, F), jnp.float32),
        ],
    )(node_scalar, node_vector, ps, pv0, pv1, pv2)


def kernel(node_scalar, node_vector, edge_index, edge_diff, edge_dist,
           W1, b1, W2, b2, Wr, br):
    src1d = edge_index[0]
    dst1d = edge_index[1]

    a123 = _mlp(node_scalar, W1, b1, W2, b2)
    r1, r2, r3, s0, s1, s2 = _rbf(edge_dist, edge_diff, Wr, br)
    rstd = [s0.reshape(E), s1.reshape(E), s2.reshape(E)]

    nvs = [node_vector[:, 0, :], node_vector[:, 1, :], node_vector[:, 2, :]]

    c1, c3, ps = _cspass(src1d, dst1d, a123, r1, r2, r3)
    pv = [_vpass(src1d, dst1d, nvs[d], c1, c3, rstd[d]) for d in range(3)]

    out_s, out_v = _final(node_scalar, node_vector, ps, pv[0], pv[1], pv[2])
    return (out_s, out_v)


# speedup vs baseline: 1.1750x; 1.1684x over previous
import functools

import jax
import jax.numpy as jnp
from jax import lax
from jax.experimental import pallas as pl
from jax.experimental.pallas import tpu as pltpu
from jax.experimental.pallas import tpu_sc as plsc

N = 10000
E = 320000
F = 128
NRBF = 20
CUTOFF = 5.0

NC = 2
NS = 16
NW = NC * NS
B = 32
EPW = E // NW
MCPW = EPW // B
PAIRS = MCPW // 2
TAIL = EPW - MCPW * B
MAIN = NW * MCPW * B
ACC_N = 10240
RPT = ACC_N // NS


def _mlp_body(x_ref, w1_ref, b1_ref, w2_ref, b2_ref, o13_ref, o2_ref):
    h = jnp.dot(x_ref[...], w1_ref[...], preferred_element_type=jnp.float32)
    h = h + b1_ref[...]
    h = h * jax.nn.sigmoid(h)
    y = jnp.dot(h, w2_ref[...], preferred_element_type=jnp.float32) + b2_ref[...]
    o13_ref[...] = jnp.concatenate([y[:, :F], y[:, 2 * F:]], axis=1)
    o2_ref[...] = y[:, F:2 * F]


def _mlp(node_scalar, W1, b1, W2, b2):
    BN = 1000
    grid = (N // BN,)
    return pl.pallas_call(
        _mlp_body,
        grid=grid,
        in_specs=[
            pl.BlockSpec((BN, F), lambda i: (i, 0)),
            pl.BlockSpec((F, F), lambda i: (0, 0)),
            pl.BlockSpec((1, F), lambda i: (0, 0)),
            pl.BlockSpec((F, 3 * F), lambda i: (0, 0)),
            pl.BlockSpec((1, 3 * F), lambda i: (0, 0)),
        ],
        out_specs=[
            pl.BlockSpec((BN, 2 * F), lambda i: (i, 0)),
            pl.BlockSpec((BN, F), lambda i: (i, 0)),
        ],
        out_shape=[
            jax.ShapeDtypeStruct((N, 2 * F), jnp.float32),
            jax.ShapeDtypeStruct((N, F), jnp.float32),
        ],
    )(node_scalar, W1, b1.reshape(1, F), W2, b2.reshape(1, 3 * F))


def _rbf_body(dist_ref, diff_ref, wr_ref, br_ref,
              o1_ref, o2_ref, d0_ref, d1_ref, d2_ref):
    r = dist_ref[...]
    n_vals = lax.broadcasted_iota(jnp.int32, (1, NRBF), 1).astype(jnp.float32) + 1.0
    facs = jnp.concatenate(
        [n_vals * (jnp.pi / CUTOFF),
         jnp.full((1, 1), jnp.pi / CUTOFF, jnp.float32)], axis=1)
    offs = jnp.concatenate(
        [jnp.zeros((1, NRBF), jnp.float32),
         jnp.full((1, 1), 0.5 * jnp.pi, jnp.float32)], axis=1)
    sall = jnp.sin(r * facs + offs)
    sinc = sall[:, :NRBF] / r
    cosq = sall[:, NRBF:]
    rbf = jnp.dot(sinc, wr_ref[...], preferred_element_type=jnp.float32)
    rbf = rbf + br_ref[...]
    cc = jnp.where(r < CUTOFF, 0.5 * (cosq + 1.0), 0.0)
    rbf = rbf * cc
    o1_ref[...] = rbf[:, :F]
    o2_ref[...] = rbf[:, F:2 * F]
    r3 = rbf[:, 2 * F:]
    rstd = diff_ref[...] / r
    d0_ref[...] = r3 * rstd[:, 0:1]
    d1_ref[...] = r3 * rstd[:, 1:2]
    d2_ref[...] = r3 * rstd[:, 2:3]


def _rbf(edge_dist, edge_diff, Wr, br):
    BE = 4000
    grid = (E // BE,)
    out = jax.ShapeDtypeStruct((E, F), jnp.float32)
    return pl.pallas_call(
        _rbf_body,
        grid=grid,
        in_specs=[
            pl.BlockSpec((BE, 1), lambda i: (i, 0)),
            pl.BlockSpec((BE, 3), lambda i: (i, 0)),
            pl.BlockSpec((NRBF, 3 * F), lambda i: (0, 0)),
            pl.BlockSpec((1, 3 * F), lambda i: (0, 0)),
        ],
        out_specs=[pl.BlockSpec((BE, F), lambda i: (i, 0))] * 5,
        out_shape=[out] * 5,
    )(edge_dist.reshape(E, 1), edge_diff, Wr, br.reshape(1, 3 * F))


_SC_MESH = plsc.VectorSubcoreMesh(core_axis_name="c", subcore_axis_name="s")


def _zero_rows(buf, nrows):
    zrow = jnp.zeros((16,), jnp.float32)

    def zbody(i, _):
        for g in range(F // 16):
            buf[i, pl.ds(g * 16, 16)] = zrow
        return 0

    lax.fori_loop(0, nrows, zbody, 0)


def _zero_acc(buf, acc, s):
    _zero_rows(buf, B)
    base = s * RPT
    for k in range(RPT // B):
        pltpu.sync_copy(buf, acc.at[pl.ds(base + k * B, B)])


def _flush_acc(buf, acc, out, c, s):
    base = s * RPT
    for k in range(RPT // B):
        pltpu.sync_copy(acc.at[pl.ds(base + k * B, B)], buf)
        pltpu.sync_copy(buf, out.at[c, pl.ds(base + k * B, B)])


def _edge_pipeline(src1d, dst1d, sets, isb, tails, sems, combine, acc, wid):
    base = wid * (MCPW * B)

    def issue_idx(p, e0):
        pltpu.async_copy(src1d.at[pl.ds(e0, B)], isb[p], sems["i"][p])

    def wait_idx(p, e0):
        pltpu.make_async_copy(src1d.at[pl.ds(e0, B)], isb[p], sems["i"][p]).wait()

    def issue_set(p, e0):
        st = sets[p]
        pltpu.async_copy(dst1d.at[pl.ds(e0, B)], st["id"], sems["d"][p])
        for tab, buf in st["gath"]:
            pltpu.async_copy(tab.at[isb[p]], buf, sems["d"][p])
        for lin, buf in st["lin"]:
            pltpu.async_copy(lin.at[pl.ds(e0, B)], buf, sems["d"][p])

    def wait_set(p, e0):
        st = sets[p]
        pltpu.make_async_copy(dst1d.at[pl.ds(e0, B)], st["id"], sems["d"][p]).wait()
        for tab, buf in st["gath"]:
            pltpu.make_async_copy(tab.at[isb[p]], buf, sems["d"][p]).wait()
        for lin, buf in st["lin"]:
            pltpu.make_async_copy(lin.at[pl.ds(e0, B)], buf,
                                  sems["d"][p]).wait()

    def issue_scatter(p):
        st = sets[p]
        pltpu.async_copy(st["msg"], acc.at[st["id"]], sems["s"][p], add=True)

    def wait_scatter(p):
        st = sets[p]
        pltpu.make_async_copy(st["msg"], acc.at[st["id"]], sems["s"][p]).wait()

    pltpu.sync_copy(src1d.at[pl.ds(base, B)], isb[0])
    issue_set(0, base)
    issue_idx(1, base + B)

    def pair(jj, _):
        e0 = base + jj * (2 * B)
        e1 = e0 + B
        e2 = e1 + B
        e3 = e2 + B
        wait_idx(1, e1)

        @pl.when(jj > 0)
        def _():
            wait_scatter(1)

        issue_set(1, e1)
        wait_set(0, e0)
        issue_idx(0, e2)
        combine(0, B)
        issue_scatter(0)
        wait_idx(0, e2)
        wait_scatter(0)
        issue_set(0, e2)
        wait_set(1, e1)
        issue_idx(1, e3)
        combine(1, B)
        issue_scatter(1)
        return 0

    lax.fori_loop(0, PAIRS, pair, 0)

    wait_set(0, base + MCPW * B)
    wait_idx(1, base + (MCPW + 1) * B)
    wait_scatter(1)

    its, itd = tails
    e0t = MAIN + wid * TAIL
    st = sets[0]
    pltpu.sync_copy(src1d.at[pl.ds(e0t, TAIL)], its)
    pltpu.sync_copy(dst1d.at[pl.ds(e0t, TAIL)], itd)
    for tab, buf in st["gath"]:
        pltpu.sync_copy(tab.at[its], buf.at[pl.ds(0, TAIL)])
    for lin, buf in st["lin"]:
        pltpu.sync_copy(lin.at[pl.ds(e0t, TAIL)], buf.at[pl.ds(0, TAIL)])
    combine(0, TAIL)
    pltpu.sync_copy(st["msg"].at[pl.ds(0, TAIL)], acc.at[itd], add=True)


@functools.partial(
    pl.kernel,
    out_type=jax.ShapeDtypeStruct((NC, ACC_N, F), jnp.float32),
    mesh=_SC_MESH,
    scratch_types=[
        pltpu.VMEM((B,), jnp.int32), pltpu.VMEM((B,), jnp.int32),
        pltpu.VMEM((B,), jnp.int32), pltpu.VMEM((B,), jnp.int32),
        pltpu.VMEM((TAIL,), jnp.int32), pltpu.VMEM((TAIL,), jnp.int32),
        pltpu.VMEM((B, F), jnp.float32), pltpu.VMEM((B, F), jnp.float32),
        pltpu.VMEM((B, F), jnp.float32), pltpu.VMEM((B, F), jnp.float32),
        pltpu.VMEM_SHARED((ACC_N, F), jnp.float32),
        pltpu.SemaphoreType.DMA, pltpu.SemaphoreType.DMA,
        pltpu.SemaphoreType.DMA, pltpu.SemaphoreType.DMA,
        pltpu.SemaphoreType.DMA, pltpu.SemaphoreType.DMA,
    ],
)
def _spass(src1d, dst1d, tab2, lin2, out,
           isa, isbb, ida, idb, its, itd, ga, gb, la, lb, acc,
           sda, sdb, sia, sib, ssa, ssb):
    c = lax.axis_index("c")
    s = lax.axis_index("s")
    wid = c * NS + s

    _zero_acc(ga, acc, s)
    plsc.subcore_barrier()

    sets = [
        {"id": ida, "gath": [(tab2, ga)], "lin": [(lin2, la)], "msg": ga},
        {"id": idb, "gath": [(tab2, gb)], "lin": [(lin2, lb)], "msg": gb},
    ]
    sems = {"d": [sda, sdb], "i": [sia, sib], "s": [ssa, ssb]}
    gbufs = [(ga, la), (gb, lb)]

    def combine(p, nrows):
        g, l = gbufs[p]

        def rbody(i, _):
            for gi in range(F // 16):
                sl = pl.ds(gi * 16, 16)
                g[i, sl] = g[i, sl] * l[i, sl]
            return 0

        lax.fori_loop(0, nrows, rbody, 0)

    _edge_pipeline(src1d, dst1d, sets, [isa, isbb], (its, itd), sems,
                   combine, acc, wid)
    plsc.subcore_barrier()
    _flush_acc(ga, acc, out, c, s)


@functools.partial(
    pl.kernel,
    out_type=jax.ShapeDtypeStruct((NC, ACC_N, F), jnp.float32),
    mesh=_SC_MESH,
    scratch_types=[
        pltpu.VMEM((B,), jnp.int32), pltpu.VMEM((B,), jnp.int32),
        pltpu.VMEM((B,), jnp.int32), pltpu.VMEM((B,), jnp.int32),
        pltpu.VMEM((TAIL,), jnp.int32), pltpu.VMEM((TAIL,), jnp.int32),
        pltpu.VMEM((B, 2 * F), jnp.float32), pltpu.VMEM((B, 2 * F), jnp.float32),
        pltpu.VMEM((B, F), jnp.float32), pltpu.VMEM((B, F), jnp.float32),
        pltpu.VMEM((B, F), jnp.float32), pltpu.VMEM((B, F), jnp.float32),
        pltpu.VMEM((B, F), jnp.float32), pltpu.VMEM((B, F), jnp.float32),
        pltpu.VMEM_SHARED((ACC_N, F), jnp.float32),
        pltpu.SemaphoreType.DMA, pltpu.SemaphoreType.DMA,
        pltpu.SemaphoreType.DMA, pltpu.SemaphoreType.DMA,
        pltpu.SemaphoreType.DMA, pltpu.SemaphoreType.DMA,
    ],
)
def _vpass(src1d, dst1d, tab13, tabn, lin1, lin3, out,
           isa, isbb, ida, idb, its, itd,
           g13a, g13b, gna, gnb, l1a, l1b, l3a, l3b, acc,
           sda, sdb, sia, sib, ssa, ssb):
    c = lax.axis_index("c")
    s = lax.axis_index("s")
    wid = c * NS + s

    _zero_acc(gna, acc, s)
    plsc.subcore_barrier()

    sets = [
        {"id": ida, "gath": [(tab13, g13a), (tabn, gna)],
         "lin": [(lin1, l1a), (lin3, l3a)], "msg": gna},
        {"id": idb, "gath": [(tab13, g13b), (tabn, gnb)],
         "lin": [(lin1, l1b), (lin3, l3b)], "msg": gnb},
    ]
    sems = {"d": [sda, sdb], "i": [sia, sib], "s": [ssa, ssb]}
    gbufs = [(g13a, gna, l1a, l3a), (g13b, gnb, l1b, l3b)]

    def combine(p, nrows):
        g13, gn, l1, l3 = gbufs[p]

        def rbody(i, _):
            for gi in range(F // 16):
                sl = pl.ds(gi * 16, 16)
                sh = pl.ds(F + gi * 16, 16)
                gn[i, sl] = gn[i, sl] * (g13[i, sl] * l1[i, sl]) \
                    + g13[i, sh] * l3[i, sl]
            return 0

        lax.fori_loop(0, nrows, rbody, 0)

    _edge_pipeline(src1d, dst1d, sets, [isa, isbb], (its, itd), sems,
                   combine, acc, wid)
    plsc.subcore_barrier()
    _flush_acc(gna, acc, out, c, s)


def _final_body(ns_ref, nv_ref, ps_ref, p0_ref, p1_ref, p2_ref,
                os_ref, ov_ref):
    os_ref[...] = ns_ref[...] + ps_ref[0] + ps_ref[1]
    dv0 = p0_ref[0] + p0_ref[1]
    dv1 = p1_ref[0] + p1_ref[1]
    dv2 = p2_ref[0] + p2_ref[1]
    ov_ref[...] = nv_ref[...] + jnp.stack([dv0, dv1, dv2], axis=1)


def _final(node_scalar, node_vector, ps, pv0, pv1, pv2):
    BN = 1000
    grid = (N // BN,)
    part_spec = pl.BlockSpec((NC, BN, F), lambda i: (0, i, 0))
    return pl.pallas_call(
        _final_body,
        grid=grid,
        in_specs=[
            pl.BlockSpec((BN, F), lambda i: (i, 0)),
            pl.BlockSpec((BN, 3, F), lambda i: (i, 0, 0)),
            part_spec, part_spec, part_spec, part_spec,
        ],
        out_specs=[
            pl.BlockSpec((BN, F), lambda i: (i, 0)),
            pl.BlockSpec((BN, 3, F), lambda i: (i, 0, 0)),
        ],
        out_shape=[
            jax.ShapeDtypeStruct((N, F), jnp.float32),
            jax.ShapeDtypeStruct((N, 3, F), jnp.float32),
        ],
    )(node_scalar, node_vector, ps, pv0, pv1, pv2)


def kernel(node_scalar, node_vector, edge_index, edge_diff, edge_dist,
           W1, b1, W2, b2, Wr, br):
    src1d = edge_index[0]
    dst1d = edge_index[1]

    a13, a2 = _mlp(node_scalar, W1, b1, W2, b2)
    rbf1, rbf2, r3d0, r3d1, r3d2 = _rbf(edge_dist, edge_diff, Wr, br)

    nv0 = node_vector[:, 0, :]
    nv1 = node_vector[:, 1, :]
    nv2 = node_vector[:, 2, :]

    ps = _spass(src1d, dst1d, a2, rbf2)
    pv0 = _vpass(src1d, dst1d, a13, nv0, rbf1, r3d0)
    pv1 = _vpass(src1d, dst1d, a13, nv1, rbf1, r3d1)
    pv2 = _vpass(src1d, dst1d, a13, nv2, rbf1, r3d2)

    out_s, out_v = _final(node_scalar, node_vector, ps, pv0, pv1, pv2)
    return (out_s, out_v)
